# baseline (speedup 1.0000x reference)
import jax
import jax.numpy as jnp
import numpy as np
from jax.experimental import pallas as pl

B = 4; L = 512; C = 7; PRED = 96
D = 128; DFF = 256; E = 8
N = B * C
OUT_W = PRED * C


def _fused_kernel(x_enc_ref, xt_raw_ref, W_in_ref, Wcat_ref, W1_ref, W2_ref,
                  out_ref):
    f32 = jnp.float32
    Wq_ = Wcat_ref[:, 0:D]
    Wk_ = Wcat_ref[:, D:2 * D]
    Wv_ = Wcat_ref[:, 2 * D:3 * D]
    Wo_ = Wcat_ref[:, 3 * D:4 * D]
    Wr_ = Wcat_ref[:, 4 * D:4 * D + E]
    Wout_ = Wcat_ref[:, 4 * D + E:4 * D + E + PRED]
    x_enc = x_enc_ref[...]
    xt_raw = xt_raw_ref[...]

    m = jnp.mean(x_enc, axis=1)
    xc = x_enc - m[:, None, :]
    m2 = jnp.mean(xc, axis=1)
    var = jnp.mean((xc - m2[:, None, :]) ** 2, axis=1)
    stdev = jnp.sqrt(var + 1e-5)
    rstd = 1.0 / stdev

    n_i = jax.lax.broadcasted_iota(jnp.int32, (N, L), 0)
    j_i = jax.lax.broadcasted_iota(jnp.int32, (N, L), 1)
    cmap = (n_i + j_i) % C
    rn = jax.lax.broadcasted_iota(jnp.int32, (N, B), 0) // C
    rb = jax.lax.broadcasted_iota(jnp.int32, (N, B), 1)
    R = (rn == rb).astype(f32)
    M_n = jnp.dot(R, m, preferred_element_type=f32)
    S_n = jnp.dot(R, rstd, preferred_element_type=f32)
    meanmap = jnp.zeros((N, L), f32)
    rstdmap = jnp.zeros((N, L), f32)
    for c in range(C):
        sel = cmap == c
        meanmap = jnp.where(sel, M_n[:, c][:, None], meanmap)
        rstdmap = jnp.where(sel, S_n[:, c][:, None], rstdmap)
    xt = (xt_raw - meanmap) * rstdmap

    w_in = W_in_ref[...]
    qv = jnp.dot(w_in, Wq_, preferred_element_type=f32)
    kv = jnp.dot(w_in, Wk_, preferred_element_type=f32)
    vv = jnp.dot(w_in, Wv_, preferred_element_type=f32)
    u = jnp.dot(vv, Wo_, preferred_element_type=f32)
    a = jnp.sum(qv * kv) * (1.0 / np.sqrt(D))

    xl = xt[:, L - 1][:, None]
    logits = (a * xl) * xt
    lmax = jnp.max(logits, axis=1, keepdims=True)
    pexp = jnp.exp(logits - lmax)
    s = (jnp.sum(pexp * xt, axis=1, keepdims=True)
         / jnp.sum(pexp, axis=1, keepdims=True))
    hf = xl * w_in + s * u

    rlog = jnp.dot(hf, Wr_, preferred_element_type=f32)
    rmax = jnp.max(rlog, axis=1, keepdims=True)
    rexp = jnp.exp(rlog - rmax)
    rp = rexp / jnp.sum(rexp, axis=1, keepdims=True)
    m1 = jnp.max(rp, axis=1, keepdims=True)
    m2v = jnp.max(jnp.where(rp == m1, -1.0, rp), axis=1, keepdims=True)
    cw = jnp.where(rp >= m2v, rp, 0.0) / (m1 + m2v)

    moe = jnp.zeros((N, D), f32)
    for e in range(E):
        g = jnp.dot(hf, W1_ref[e], preferred_element_type=f32)
        ge = g * jax.nn.sigmoid(g)
        ye = jnp.dot(ge, W2_ref[e], preferred_element_type=f32)
        moe = moe + cw[:, e][:, None] * ye
    hff = hf + moe
    preds = jnp.dot(hff, Wout_, preferred_element_type=f32)

    bi = jax.lax.broadcasted_iota(jnp.int32, (B, N), 0)
    ni = jax.lax.broadcasted_iota(jnp.int32, (B, N), 1)
    S5 = (ni == C * bi + (C - 2)).astype(f32)
    S6 = (ni == C * bi + (C - 1)).astype(f32)
    p5 = jnp.dot(S5, preds, preferred_element_type=f32)
    p6 = jnp.dot(S6, preds, preferred_element_type=f32)
    x6 = jnp.dot(S6, xt, preferred_element_type=f32)
    a_start = L * C - (C - 2) * (L + PRED) - L
    val = jnp.concatenate([p5[:, a_start:], x6, p6], axis=1)

    ci = jax.lax.broadcasted_iota(jnp.int32, (B, OUT_W), 1) % C
    sdm = jnp.zeros((B, OUT_W), f32)
    mnm = jnp.zeros((B, OUT_W), f32)
    for c in range(C):
        sel = ci == c
        sdm = jnp.where(sel, stdev[:, c][:, None], sdm)
        mnm = jnp.where(sel, m[:, c][:, None], mnm)
    out_ref[...] = val * sdm + mnm


def kernel(x_enc, x_mark_enc, x_dec, x_mark_dec, W_in, Wq, Wk, Wv, Wo, Wr,
           W1, W2, W_out):
    xt_raw = jnp.reshape(x_enc, (N, L))
    Wcat = jnp.concatenate([Wq, Wk, Wv, Wo, Wr, W_out], axis=1)
    out = pl.pallas_call(
        _fused_kernel,
        out_shape=jax.ShapeDtypeStruct((B, OUT_W), jnp.float32),
    )(x_enc, xt_raw, W_in, Wcat, W1, W2)
    return jnp.reshape(out, (B, PRED, C))


# speedup vs baseline: 1.1597x; 1.1597x over previous
import jax
import jax.numpy as jnp
import numpy as np
from jax.experimental import pallas as pl
from jax.experimental.pallas import tpu as pltpu

B = 4; L = 512; C = 7; PRED = 96
D = 128; DFF = 256; E = 8
N = B * C
OUT_W = PRED * C


def _fused_kernel(xt_raw_ref, W_in_ref, Wq_ref, Wk_ref, Wv_ref,
                  Wo_ref, Wr_ref, W1_ref, W2_ref, W_out_ref, out_ref,
                  w1_s, w2_s, sem1, sem2):
    f32 = jnp.float32
    cp1 = pltpu.make_async_copy(W1_ref, w1_s, sem1)
    cp2 = pltpu.make_async_copy(W2_ref, w2_s, sem2)
    cp1.start()
    cp2.start()
    xt_raw = xt_raw_ref[...]

    j_k = jax.lax.broadcasted_iota(jnp.int32, (L, C), 0) % C
    r_k = jax.lax.broadcasted_iota(jnp.int32, (L, C), 1)
    Kmat = (j_k == r_k).astype(f32)
    H1 = jnp.dot(xt_raw, Kmat, preferred_element_type=f32)
    H2 = jnp.dot(xt_raw * xt_raw, Kmat, preferred_element_type=f32)
    ncc = jax.lax.broadcasted_iota(jnp.int32, (N, C), 0)
    ccc = jax.lax.broadcasted_iota(jnp.int32, (N, C), 1)
    rot = (ccc - ncc) % C
    G1 = jnp.zeros((N, C), f32)
    G2 = jnp.zeros((N, C), f32)
    for r in range(C):
        sel = rot == r
        G1 = jnp.where(sel, H1[:, r][:, None], G1)
        G2 = jnp.where(sel, H2[:, r][:, None], G2)
    bib = jax.lax.broadcasted_iota(jnp.int32, (B, N), 0)
    nib = jax.lax.broadcasted_iota(jnp.int32, (B, N), 1)
    Sb = (nib // C == bib).astype(f32)
    m = jnp.dot(Sb, G1, preferred_element_type=f32) * (1.0 / L)
    ex2 = jnp.dot(Sb, G2, preferred_element_type=f32) * (1.0 / L)
    var = ex2 - m * m
    stdev = jnp.sqrt(var + 1e-5)
    rstd = 1.0 / stdev

    NS = 2 * B
    rr = jax.lax.broadcasted_iota(jnp.int32, (NS, N), 0)
    nn = jax.lax.broadcasted_iota(jnp.int32, (NS, N), 1)
    Ssel = (nn == C * (rr // 2) + (C - 2) + (rr % 2)).astype(f32)
    xts_raw = jnp.dot(Ssel, xt_raw, preferred_element_type=f32)

    r_i = jax.lax.broadcasted_iota(jnp.int32, (NS, L), 0)
    j_i = jax.lax.broadcasted_iota(jnp.int32, (NS, L), 1)
    cmap = (C - 2 + (r_i % 2) + j_i) % C
    rb8 = jax.lax.broadcasted_iota(jnp.int32, (NS, B), 0) // 2
    bb8 = jax.lax.broadcasted_iota(jnp.int32, (NS, B), 1)
    R8 = (rb8 == bb8).astype(f32)
    M_n = jnp.dot(R8, m, preferred_element_type=f32)
    S_n = jnp.dot(R8, rstd, preferred_element_type=f32)
    meanmap = jnp.zeros((NS, L), f32)
    rstdmap = jnp.zeros((NS, L), f32)
    for c in range(C):
        sel = cmap == c
        meanmap = jnp.where(sel, M_n[:, c][:, None], meanmap)
        rstdmap = jnp.where(sel, S_n[:, c][:, None], rstdmap)
    xt = (xts_raw - meanmap) * rstdmap

    w_in = W_in_ref[...]
    qv = jnp.dot(w_in, Wq_ref[...], preferred_element_type=f32)
    kv = jnp.dot(w_in, Wk_ref[...], preferred_element_type=f32)
    vv = jnp.dot(w_in, Wv_ref[...], preferred_element_type=f32)
    u = jnp.dot(vv, Wo_ref[...], preferred_element_type=f32)
    a = jnp.sum(qv * kv) * (1.0 / np.sqrt(D))

    xl = xt[:, L - 1][:, None]
    logits = (a * xl) * xt
    lmax = jnp.max(logits, axis=1, keepdims=True)
    pexp = jnp.exp(logits - lmax)
    s = (jnp.sum(pexp * xt, axis=1, keepdims=True)
         / jnp.sum(pexp, axis=1, keepdims=True))
    hf = xl * w_in + s * u

    rlog = jnp.dot(hf, Wr_ref[...], preferred_element_type=f32)
    rmax = jnp.max(rlog, axis=1, keepdims=True)
    rexp = jnp.exp(rlog - rmax)
    rp = rexp / jnp.sum(rexp, axis=1, keepdims=True)
    m1 = jnp.max(rp, axis=1, keepdims=True)
    m2v = jnp.max(jnp.where(rp == m1, -1.0, rp), axis=1, keepdims=True)
    cw = jnp.where(rp >= m2v, rp, 0.0) / (m1 + m2v)

    cp1.wait()
    cp2.wait()
    moe = jnp.zeros((NS, D), f32)
    for e in range(E):
        g = jnp.dot(hf, w1_s[e], preferred_element_type=f32)
        ge = g * jax.nn.sigmoid(g)
        ye = jnp.dot(ge, w2_s[e], preferred_element_type=f32)
        moe = moe + cw[:, e][:, None] * ye
    hff = hf + moe
    preds = jnp.dot(hff, W_out_ref[...], preferred_element_type=f32)

    bi = jax.lax.broadcasted_iota(jnp.int32, (B, NS), 0)
    ni = jax.lax.broadcasted_iota(jnp.int32, (B, NS), 1)
    S5 = (ni == 2 * bi).astype(f32)
    S6 = (ni == 2 * bi + 1).astype(f32)
    p5 = jnp.dot(S5, preds, preferred_element_type=f32)
    p6 = jnp.dot(S6, preds, preferred_element_type=f32)
    x6 = jnp.dot(S6, xt, preferred_element_type=f32)
    a_start = L * C - (C - 2) * (L + PRED) - L
    val = jnp.concatenate([p5[:, a_start:], x6, p6], axis=1)

    ci = jax.lax.broadcasted_iota(jnp.int32, (B, OUT_W), 1) % C
    sdm = jnp.zeros((B, OUT_W), f32)
    mnm = jnp.zeros((B, OUT_W), f32)
    for c in range(C):
        sel = ci == c
        sdm = jnp.where(sel, stdev[:, c][:, None], sdm)
        mnm = jnp.where(sel, m[:, c][:, None], mnm)
    out_ref[...] = val * sdm + mnm


def kernel(x_enc, x_mark_enc, x_dec, x_mark_dec, W_in, Wq, Wk, Wv, Wo, Wr,
           W1, W2, W_out):
    xt_raw = jnp.reshape(x_enc, (N, L))
    vmem = pl.BlockSpec(memory_space=pltpu.MemorySpace.VMEM)
    hbm = pl.BlockSpec(memory_space=pltpu.MemorySpace.HBM)
    out = pl.pallas_call(
        _fused_kernel,
        in_specs=[vmem, vmem, vmem, vmem, vmem, vmem, vmem,
                  hbm, hbm, vmem],
        out_specs=vmem,
        out_shape=jax.ShapeDtypeStruct((B, OUT_W), jnp.float32),
        scratch_shapes=[
            pltpu.VMEM((E, D, DFF), jnp.float32),
            pltpu.VMEM((E, DFF, D), jnp.float32),
            pltpu.SemaphoreType.DMA,
            pltpu.SemaphoreType.DMA,
        ],
    )(xt_raw, W_in, Wq, Wk, Wv, Wo, Wr, W1, W2, W_out)
    return jnp.reshape(out, (B, PRED, C))
